# initial kernel scaffold (unmeasured)
import jax
import jax.numpy as jnp
from jax import lax
from jax.experimental import pallas as pl
from jax.experimental.pallas import tpu as pltpu

N_DEV = 8
E_LOCAL = 8


def kernel(x, router_W, route_idx, expert_W, shared_W):
    n, d = x.shape
    _, h = shared_W.shape
    rows = n // N_DEV

    def body(x_ref, rw_ref, idx_ref, ew_ref, sw_ref, out_ref,
             acc_ref, send_ref, recv_ref, send_sem, recv_sems):
        my = lax.axis_index("i")
        left = lax.rem(my + N_DEV - 1, N_DEV)
        right = lax.rem(my + 1, N_DEV)

        barrier_sem = pltpu.get_barrier_semaphore()
        for nbr in (left, right):
            pl.semaphore_signal(
                barrier_sem, inc=1,
                device_id=(nbr,), device_id_type=pl.DeviceIdType.MESH,
            )
        pl.semaphore_wait(barrier_sem, 2)

        xv = x_ref[...]
        scores = jnp.dot(xv, rw_ref[...], preferred_element_type=jnp.float32)
        scores = scores - jnp.max(scores, axis=1, keepdims=True)
        e_sc = jnp.exp(scores)
        probs = e_sc / jnp.sum(e_sc, axis=1, keepdims=True)
        ids = idx_ref[...]
        eids = lax.broadcasted_iota(jnp.int32, probs.shape, 1)
        p_tok = jnp.sum(jnp.where(eids == ids, probs, 0.0),
                        axis=1, keepdims=True)

        for j in range(E_LOCAL):
            e = my * E_LOCAL + j
            w = jnp.where(ids == e, p_tok, 0.0)
            g = jnp.dot(xv * w, ew_ref[j],
                        preferred_element_type=jnp.float32)
            if j == 0:
                acc_ref[...] = g
            else:
                acc_ref[...] += g

        for s in range(N_DEV - 1):
            c = lax.rem(my + 2 * N_DEV - 1 - s, N_DEV)
            chunk = acc_ref[pl.ds(c * rows, rows), :]
            if s > 0:
                chunk = chunk + recv_ref[s - 1]
            send_ref[...] = chunk
            rdma = pltpu.make_async_remote_copy(
                src_ref=send_ref,
                dst_ref=recv_ref.at[s],
                send_sem=send_sem,
                recv_sem=recv_sems.at[s],
                device_id=(right,),
                device_id_type=pl.DeviceIdType.MESH,
            )
            rdma.start()
            rdma.wait()

        xs = x_ref[pl.ds(my * rows, rows), :]
        shared = jnp.dot(xs, sw_ref[...], preferred_element_type=jnp.float32)
        out_ref[...] = (shared + acc_ref[pl.ds(my * rows, rows), :]
                        + recv_ref[N_DEV - 2])

    return pl.pallas_call(
        body,
        out_shape=jax.ShapeDtypeStruct((rows, h), jnp.float32),
        in_specs=[pl.BlockSpec(memory_space=pltpu.VMEM)] * 5,
        out_specs=pl.BlockSpec(memory_space=pltpu.VMEM),
        scratch_shapes=[
            pltpu.VMEM((n, h), jnp.float32),
            pltpu.VMEM((rows, h), jnp.float32),
            pltpu.VMEM((N_DEV - 1, rows, h), jnp.float32),
            pltpu.SemaphoreType.DMA,
            pltpu.SemaphoreType.DMA((N_DEV - 1,)),
        ],
        compiler_params=pltpu.CompilerParams(collective_id=0),
    )(x, router_W, route_idx, expert_W, shared_W)


# baseline (device time: 132132 ns/iter reference)
import jax
import jax.numpy as jnp
from jax import lax
from jax.experimental import pallas as pl
from jax.experimental.pallas import tpu as pltpu

N_DEV = 8
E_LOCAL = 8


def kernel(x, router_W, route_idx, expert_W, shared_W):
    n, d = x.shape
    _, h = shared_W.shape
    rows = n // N_DEV

    def body(x_ref, rw_ref, idx_ref, ew_ref, sw_ref, out_ref,
             acc_ref, send_ref, recv_ref, send_sem, recv_sems):
        my = lax.axis_index("i")
        left = lax.rem(my + N_DEV - 1, N_DEV)
        right = lax.rem(my + 1, N_DEV)

        barrier_sem = pltpu.get_barrier_semaphore()
        for nbr in (left, right):
            pl.semaphore_signal(
                barrier_sem, inc=1,
                device_id=(nbr,), device_id_type=pl.DeviceIdType.MESH,
            )
        pl.semaphore_wait(barrier_sem, 2)

        xv = x_ref[...]
        scores = jnp.dot(xv, rw_ref[...], preferred_element_type=jnp.float32)
        scores = scores - jnp.max(scores, axis=1, keepdims=True)
        e_sc = jnp.exp(scores)
        probs = e_sc / jnp.sum(e_sc, axis=1, keepdims=True)
        ids = idx_ref[...]
        eids = lax.broadcasted_iota(jnp.int32, probs.shape, 1)
        p_tok = jnp.sum(jnp.where(eids == ids, probs, 0.0),
                        axis=1, keepdims=True)

        for j in range(E_LOCAL):
            e = my * E_LOCAL + j
            w = jnp.where(ids == e, p_tok, 0.0)
            g = jnp.dot(xv * w, ew_ref[j],
                        preferred_element_type=jnp.float32)
            if j == 0:
                acc_ref[...] = g
            else:
                acc_ref[...] += g

        for s in range(N_DEV - 1):
            c = lax.rem(my + 2 * N_DEV - 1 - s, N_DEV)
            chunk = acc_ref[pl.ds(c * rows, rows), :]
            if s > 0:
                chunk = chunk + recv_ref[s - 1]
            send_ref[...] = chunk
            rdma = pltpu.make_async_remote_copy(
                src_ref=send_ref,
                dst_ref=recv_ref.at[s],
                send_sem=send_sem,
                recv_sem=recv_sems.at[s],
                device_id=(right,),
                device_id_type=pl.DeviceIdType.MESH,
            )
            rdma.start()
            rdma.wait()

        xs = x_ref[pl.ds(my * rows, rows), :]
        shared = jnp.dot(xs, sw_ref[...], preferred_element_type=jnp.float32)
        out_ref[...] = (shared + acc_ref[pl.ds(my * rows, rows), :]
                        + recv_ref[N_DEV - 2])

    return pl.pallas_call(
        body,
        out_shape=jax.ShapeDtypeStruct((rows, h), jnp.float32),
        in_specs=[pl.BlockSpec(memory_space=pltpu.VMEM)] * 5,
        out_specs=pl.BlockSpec(memory_space=pltpu.VMEM),
        scratch_shapes=[
            pltpu.VMEM((n, h), jnp.float32),
            pltpu.VMEM((rows, h), jnp.float32),
            pltpu.VMEM((N_DEV - 1, rows, h), jnp.float32),
            pltpu.SemaphoreType.DMA,
            pltpu.SemaphoreType.DMA((N_DEV - 1,)),
        ],
        compiler_params=pltpu.CompilerParams(
            collective_id=0, vmem_limit_bytes=100 * 1024 * 1024
        ),
    )(x, router_W, route_idx, expert_W, shared_W)


# device time: 117662 ns/iter; 1.1230x vs baseline; 1.1230x over previous
import jax
import jax.numpy as jnp
from jax import lax
from jax.experimental import pallas as pl
from jax.experimental.pallas import tpu as pltpu

N_DEV = 8
E_LOCAL = 8


def kernel(x, router_W, route_idx, expert_W, shared_W):
    n, d = x.shape
    _, h = shared_W.shape
    rows = n // N_DEV

    def body(x_ref, rw_ref, idx_ref, ew_ref, sw_ref, out_ref,
             ptok_ref, send_ref, recv_ref, send_sems, recv_sems):
        my = lax.axis_index("i")
        left = lax.rem(my + N_DEV - 1, N_DEV)
        right = lax.rem(my + 1, N_DEV)

        barrier_sem = pltpu.get_barrier_semaphore()
        for nbr in (left, right):
            pl.semaphore_signal(
                barrier_sem, inc=1,
                device_id=(nbr,), device_id_type=pl.DeviceIdType.MESH,
            )
        pl.semaphore_wait(barrier_sem, 2)

        xv = x_ref[...]
        scores = jnp.dot(xv, rw_ref[...], preferred_element_type=jnp.float32)
        scores = scores - jnp.max(scores, axis=1, keepdims=True)
        e_sc = jnp.exp(scores)
        probs = e_sc / jnp.sum(e_sc, axis=1, keepdims=True)
        ids = idx_ref[...]
        eids = lax.broadcasted_iota(jnp.int32, probs.shape, 1)
        ptok_ref[...] = jnp.sum(jnp.where(eids == ids, probs, 0.0),
                                axis=1, keepdims=True)

        def chunk_partial(c):
            xc = x_ref[pl.ds(c * rows, rows), :]
            idc = idx_ref[pl.ds(c * rows, rows), :]
            pc = ptok_ref[pl.ds(c * rows, rows), :]
            acc = None
            for j in range(E_LOCAL):
                e = my * E_LOCAL + j
                w = jnp.where(idc == e, pc, 0.0)
                g = jnp.dot(xc * w, ew_ref[j],
                            preferred_element_type=jnp.float32)
                acc = g if acc is None else acc + g
            return acc

        def mk(src_slot, hop):
            return pltpu.make_async_remote_copy(
                src_ref=send_ref.at[src_slot],
                dst_ref=recv_ref.at[hop],
                send_sem=send_sems.at[src_slot],
                recv_sem=recv_sems.at[hop],
                device_id=(right,),
                device_id_type=pl.DeviceIdType.MESH,
            )

        rdmas = []
        send_ref[0] = chunk_partial(lax.rem(my + N_DEV - 1, N_DEV))
        rdmas.append(mk(0, 0))
        rdmas[0].start()

        for s in range(1, N_DEV - 1):
            c = lax.rem(my + 2 * N_DEV - 1 - s, N_DEV)
            part = chunk_partial(c)
            rdmas[s - 1].wait_recv()
            slot = s % 2
            if s >= 2:
                rdmas[s - 2].wait_send()
            send_ref[slot] = part + recv_ref[s - 1]
            rdmas.append(mk(slot, s))
            rdmas[s].start()

        own = chunk_partial(my)
        xs = x_ref[pl.ds(my * rows, rows), :]
        shared = jnp.dot(xs, sw_ref[...], preferred_element_type=jnp.float32)
        rdmas[N_DEV - 2].wait_recv()
        out_ref[...] = own + shared + recv_ref[N_DEV - 2]
        rdmas[N_DEV - 3].wait_send()
        rdmas[N_DEV - 2].wait_send()

    return pl.pallas_call(
        body,
        out_shape=jax.ShapeDtypeStruct((rows, h), jnp.float32),
        in_specs=[pl.BlockSpec(memory_space=pltpu.VMEM)] * 5,
        out_specs=pl.BlockSpec(memory_space=pltpu.VMEM),
        scratch_shapes=[
            pltpu.VMEM((n, 1), jnp.float32),
            pltpu.VMEM((2, rows, h), jnp.float32),
            pltpu.VMEM((N_DEV - 1, rows, h), jnp.float32),
            pltpu.SemaphoreType.DMA((2,)),
            pltpu.SemaphoreType.DMA((N_DEV - 1,)),
        ],
        compiler_params=pltpu.CompilerParams(
            collective_id=0, vmem_limit_bytes=100 * 1024 * 1024
        ),
    )(x, router_W, route_idx, expert_W, shared_W)


# device time: 79382 ns/iter; 1.6645x vs baseline; 1.4822x over previous
import jax
import jax.numpy as jnp
from jax import lax
from jax.experimental import pallas as pl
from jax.experimental.pallas import tpu as pltpu

N_DEV = 8
E_LOCAL = 8


def kernel(x, router_W, route_idx, expert_W, shared_W):
    n, d = x.shape
    _, h = shared_W.shape
    rows = n // N_DEV
    h2 = h // 2

    def body(x_ref, rw_ref, idx_ref, ew_ref, sw_ref, out_ref, ptok_ref,
             sendr_ref, sendl_ref, recvr_ref, recvl_ref,
             sendr_sems, sendl_sems, recvr_sems, recvl_sems):
        my = lax.axis_index("i")
        left = lax.rem(my + N_DEV - 1, N_DEV)
        right = lax.rem(my + 1, N_DEV)

        barrier_sem = pltpu.get_barrier_semaphore()
        for nbr in (left, right):
            pl.semaphore_signal(
                barrier_sem, inc=1,
                device_id=(nbr,), device_id_type=pl.DeviceIdType.MESH,
            )
        pl.semaphore_wait(barrier_sem, 2)

        xv = x_ref[...]
        scores = jnp.dot(xv, rw_ref[...], preferred_element_type=jnp.float32)
        scores = scores - jnp.max(scores, axis=1, keepdims=True)
        e_sc = jnp.exp(scores)
        probs = e_sc / jnp.sum(e_sc, axis=1, keepdims=True)
        ids = idx_ref[...]
        eids = lax.broadcasted_iota(jnp.int32, probs.shape, 1)
        ptok_ref[...] = jnp.sum(jnp.where(eids == ids, probs, 0.0),
                                axis=1, keepdims=True)

        def half_partial(c, half):
            xc = x_ref[pl.ds(c * rows, rows), :]
            idc = idx_ref[pl.ds(c * rows, rows), :]
            pc = ptok_ref[pl.ds(c * rows, rows), :]
            acc = None
            for j in range(E_LOCAL):
                e = my * E_LOCAL + j
                w = jnp.where(idc == e, pc, 0.0)
                g = jnp.dot(xc * w, ew_ref[j, :, pl.ds(half * h2, h2)],
                            preferred_element_type=jnp.float32)
                acc = g if acc is None else acc + g
            return acc

        def mk(send_ref, recv_ref, send_sems, recv_sems, slot, hop, dst):
            return pltpu.make_async_remote_copy(
                src_ref=send_ref.at[slot],
                dst_ref=recv_ref.at[hop],
                send_sem=send_sems.at[slot],
                recv_sem=recv_sems.at[hop],
                device_id=(dst,),
                device_id_type=pl.DeviceIdType.MESH,
            )

        def mkr(slot, hop):
            return mk(sendr_ref, recvr_ref, sendr_sems, recvr_sems,
                      slot, hop, right)

        def mkl(slot, hop):
            return mk(sendl_ref, recvl_ref, sendl_sems, recvl_sems,
                      slot, hop, left)

        sendr_ref[0] = half_partial(lax.rem(my + N_DEV - 1, N_DEV), 0)
        sendl_ref[0] = half_partial(lax.rem(my + 1, N_DEV), 1)
        rr = [mkr(0, 0)]
        rl = [mkl(0, 0)]
        rr[0].start()
        rl[0].start()

        for s in range(1, N_DEV - 1):
            cr = lax.rem(my + 2 * N_DEV - 1 - s, N_DEV)
            cl = lax.rem(my + 1 + s, N_DEV)
            pr = half_partial(cr, 0)
            pldat = half_partial(cl, 1)
            rr[s - 1].wait_recv()
            rl[s - 1].wait_recv()
            slot = s % 2
            if s >= 2:
                rr[s - 2].wait_send()
                rl[s - 2].wait_send()
            sendr_ref[slot] = pr + recvr_ref[s - 1]
            sendl_ref[slot] = pldat + recvl_ref[s - 1]
            rr.append(mkr(slot, s))
            rl.append(mkl(slot, s))
            rr[s].start()
            rl[s].start()

        own_r = half_partial(my, 0)
        own_l = half_partial(my, 1)
        xs = x_ref[pl.ds(my * rows, rows), :]
        shared = jnp.dot(xs, sw_ref[...], preferred_element_type=jnp.float32)
        rr[N_DEV - 2].wait_recv()
        rl[N_DEV - 2].wait_recv()
        out_ref[:, :h2] = own_r + shared[:, :h2] + recvr_ref[N_DEV - 2]
        out_ref[:, h2:] = own_l + shared[:, h2:] + recvl_ref[N_DEV - 2]
        for hop in (N_DEV - 3, N_DEV - 2):
            rr[hop].wait_send()
            rl[hop].wait_send()

    return pl.pallas_call(
        body,
        out_shape=jax.ShapeDtypeStruct((rows, h), jnp.float32),
        in_specs=[pl.BlockSpec(memory_space=pltpu.VMEM)] * 5,
        out_specs=pl.BlockSpec(memory_space=pltpu.VMEM),
        scratch_shapes=[
            pltpu.VMEM((n, 1), jnp.float32),
            pltpu.VMEM((2, rows, h2), jnp.float32),
            pltpu.VMEM((2, rows, h2), jnp.float32),
            pltpu.VMEM((N_DEV - 1, rows, h2), jnp.float32),
            pltpu.VMEM((N_DEV - 1, rows, h2), jnp.float32),
            pltpu.SemaphoreType.DMA((2,)),
            pltpu.SemaphoreType.DMA((2,)),
            pltpu.SemaphoreType.DMA((N_DEV - 1,)),
            pltpu.SemaphoreType.DMA((N_DEV - 1,)),
        ],
        compiler_params=pltpu.CompilerParams(
            collective_id=0, vmem_limit_bytes=100 * 1024 * 1024
        ),
    )(x, router_W, route_idx, expert_W, shared_W)


# device time: 69066 ns/iter; 1.9131x vs baseline; 1.1494x over previous
import jax
import jax.numpy as jnp
from jax import lax
from jax.experimental import pallas as pl
from jax.experimental.pallas import tpu as pltpu

N_DEV = 8
E_LOCAL = 8

SECS = ((0, 384, (0, 1, 2)), (384, 384, (1, 2, 0)), (768, 256, (2, 0, 1)))


def _xor(a, b):
    return a + b - 2 * a * b


def _pos(bx, by, bz):
    return bz * 4 + 2 * by + _xor(bx, by)


def kernel(x, router_W, route_idx, expert_W, shared_W):
    n, d = x.shape
    _, h = shared_W.shape
    rows = n // N_DEV

    def body(x_ref, rw_ref, idx_ref, ew_ref, sw_ref, out_ref, ptok_ref,
             acc_ref, r1_ref, r2_ref, r3_ref, send_sems, recv_sems):
        p = lax.axis_index("i")
        mz = p // 4
        q = lax.rem(p, 4)
        myy = q // 2
        mx = _xor(lax.rem(q, 2), myy)
        mb = (mx, myy, mz)

        def flipped(dim):
            bits = list(mb)
            bits[dim] = 1 - bits[dim]
            return _pos(*bits)

        partner = [flipped(0), flipped(1), flipped(2)]

        barrier_sem = pltpu.get_barrier_semaphore()
        for dim in range(3):
            pl.semaphore_signal(
                barrier_sem, inc=1,
                device_id=(partner[dim],),
                device_id_type=pl.DeviceIdType.MESH,
            )
        pl.semaphore_wait(barrier_sem, 3)

        xv = x_ref[...]
        scores = jnp.dot(xv, rw_ref[...], preferred_element_type=jnp.float32)
        scores = scores - jnp.max(scores, axis=1, keepdims=True)
        e_sc = jnp.exp(scores)
        probs = e_sc / jnp.sum(e_sc, axis=1, keepdims=True)
        ids = idx_ref[...]
        eids = lax.broadcasted_iota(jnp.int32, probs.shape, 1)
        ptok_ref[...] = jnp.sum(jnp.where(eids == ids, probs, 0.0),
                                axis=1, keepdims=True)

        def store_partial(c, sec):
            off, w, _ = SECS[sec]
            xc = x_ref[pl.ds(c * rows, rows), :]
            idc = idx_ref[pl.ds(c * rows, rows), :]
            pc = ptok_ref[pl.ds(c * rows, rows), :]
            acc = None
            for j in range(E_LOCAL):
                e = p * E_LOCAL + j
                wgt = jnp.where(idc == e, pc, 0.0)
                g = jnp.dot(xc * wgt, ew_ref[j, :, pl.ds(off, w)],
                            preferred_element_type=jnp.float32)
                acc = g if acc is None else acc + g
            acc_ref[pl.ds(c * rows, rows), pl.ds(off, w)] = acc

        def send_bits(sec, phase, slot):
            d1, d2, d3 = SECS[sec][2]
            b = [None] * 3
            if phase == 1:
                b[d1] = 1 - mb[d1]
                b[d2] = slot % 2
                b[d3] = slot // 2
            elif phase == 2:
                b[d1] = mb[d1]
                b[d2] = 1 - mb[d2]
                b[d3] = slot
            else:
                b[d1], b[d2], b[d3] = mb[d1], mb[d2], 1 - mb[d3]
            return b

        rv = [r1_ref, r2_ref, r3_ref]
        msgs = {}

        def start_msg(sec, phase, slot):
            off, w, order = SECS[sec]
            mid = sec * 7 + (slot if phase == 1 else 4 + slot if phase == 2
                             else 6)
            c = _pos(*send_bits(sec, phase, slot))
            desc = pltpu.make_async_remote_copy(
                src_ref=acc_ref.at[pl.ds(c * rows, rows), pl.ds(off, w)],
                dst_ref=rv[phase - 1].at[slot, :, pl.ds(off, w)],
                send_sem=send_sems.at[mid],
                recv_sem=recv_sems.at[mid],
                device_id=(partner[order[phase - 1]],),
                device_id_type=pl.DeviceIdType.MESH,
            )
            desc.start()
            msgs[(sec, phase, slot)] = desc

        def add_recv(sec, phase, slot, chunk_bits):
            off, w, _ = SECS[sec]
            c = _pos(*chunk_bits)
            val = rv[phase - 1][pl.ds(slot, 1), :, pl.ds(off, w)][0]
            cur = acc_ref[pl.ds(c * rows, rows), pl.ds(off, w)]
            acc_ref[pl.ds(c * rows, rows), pl.ds(off, w)] = cur + val

        for sec in range(3):
            for slot in range(4):
                store_partial(_pos(*send_bits(sec, 1, slot)), sec)
                start_msg(sec, 1, slot)

        for sec in range(3):
            for slot in range(2):
                store_partial(_pos(*send_bits(sec, 2, slot)), sec)

        for sec in range(3):
            d1, d2, d3 = SECS[sec][2]
            for slot in range(4):
                msgs[(sec, 1, slot)].wait_recv()
            for v in range(2):
                s_dyn = 2 * v + (1 - mb[d2])
                bits = [None] * 3
                bits[d1], bits[d2], bits[d3] = mb[d1], 1 - mb[d2], v
                add_recv(sec, 1, s_dyn, bits)
            for slot in range(2):
                start_msg(sec, 2, slot)

        for sec in range(3):
            d1, d2, d3 = SECS[sec][2]
            for v in range(2):
                bits = [None] * 3
                bits[d1], bits[d2], bits[d3] = mb[d1], mb[d2], v
                store_partial(_pos(*bits), sec)

        for sec in range(3):
            d1, d2, d3 = SECS[sec][2]
            for v in range(2):
                s_dyn = 2 * v + mb[d2]
                bits = [None] * 3
                bits[d1], bits[d2], bits[d3] = mb[d1], mb[d2], v
                add_recv(sec, 1, s_dyn, bits)

        for sec in range(3):
            d1, d2, d3 = SECS[sec][2]
            for slot in range(2):
                msgs[(sec, 2, slot)].wait_recv()
            for slot in range(2):
                bits = [None] * 3
                bits[d1], bits[d2], bits[d3] = mb[d1], mb[d2], slot
                add_recv(sec, 2, slot, bits)
            start_msg(sec, 3, 0)

        xs = x_ref[pl.ds(p * rows, rows), :]
        shared = jnp.dot(xs, sw_ref[...], preferred_element_type=jnp.float32)

        for sec in range(3):
            msgs[(sec, 3, 0)].wait_recv()
            add_recv(sec, 3, 0, list(mb))

        out_ref[...] = acc_ref[pl.ds(p * rows, rows), :] + shared

        for desc in msgs.values():
            desc.wait_send()

    return pl.pallas_call(
        body,
        out_shape=jax.ShapeDtypeStruct((rows, h), jnp.float32),
        in_specs=[pl.BlockSpec(memory_space=pltpu.VMEM)] * 5,
        out_specs=pl.BlockSpec(memory_space=pltpu.VMEM),
        scratch_shapes=[
            pltpu.VMEM((n, 1), jnp.float32),
            pltpu.VMEM((n, h), jnp.float32),
            pltpu.VMEM((4, rows, h), jnp.float32),
            pltpu.VMEM((2, rows, h), jnp.float32),
            pltpu.VMEM((1, rows, h), jnp.float32),
            pltpu.SemaphoreType.DMA((21,)),
            pltpu.SemaphoreType.DMA((21,)),
        ],
        compiler_params=pltpu.CompilerParams(
            collective_id=0, vmem_limit_bytes=100 * 1024 * 1024
        ),
    )(x, router_W, route_idx, expert_W, shared_W)
